# baseline (speedup 1.0000x reference)
import jax
import jax.numpy as jnp
from jax import lax
from jax.experimental import pallas as pl


def _feat_mm_body(x_ref, w_ref, y_ref):
    y_ref[...] = lax.dot_general(
        x_ref[...], w_ref[...], (((1,), (1,)), ((), ())),
        preferred_element_type=jnp.float32)


def _agg_body(a_ref, y_ref, b_ref, al_ref, o_ref):
    acc = lax.dot_general(
        a_ref[...], y_ref[...], (((1,), (0,)), ((), ())),
        preferred_element_type=jnp.float32)
    h = acc + b_ref[...]
    alpha = al_ref[0, 0]
    o_ref[...] = jnp.where(h >= 0.0, h, alpha * h)


def _gcn_layer(x, adj2d, w, b, alpha, bm_feat=2000, bm_agg=200):
    n, d_in = x.shape
    d_out = w.shape[0]
    y = pl.pallas_call(
        _feat_mm_body,
        grid=(n // bm_feat,),
        in_specs=[
            pl.BlockSpec((bm_feat, d_in), lambda i: (i, 0)),
            pl.BlockSpec((d_out, d_in), lambda i: (0, 0)),
        ],
        out_specs=pl.BlockSpec((bm_feat, d_out), lambda i: (i, 0)),
        out_shape=jax.ShapeDtypeStruct((n, d_out), jnp.float32),
    )(x, w)
    h = pl.pallas_call(
        _agg_body,
        grid=(n // bm_agg,),
        in_specs=[
            pl.BlockSpec((bm_agg, n), lambda i: (i, 0)),
            pl.BlockSpec((n, d_out), lambda i: (0, 0)),
            pl.BlockSpec((1, d_out), lambda i: (0, 0)),
            pl.BlockSpec((1, 1), lambda i: (0, 0)),
        ],
        out_specs=pl.BlockSpec((bm_agg, d_out), lambda i: (i, 0)),
        out_shape=jax.ShapeDtypeStruct((n, d_out), jnp.float32),
    )(adj2d, y, b.reshape(1, -1), alpha.reshape(1, 1))
    return h


def kernel(features, seq1, adj, b1, W1, a1, b2, W2, a2, sparse):
    del seq1, sparse
    x = features[0]
    adj2d = adj[0]
    h1 = _gcn_layer(x, adj2d, W1, b1, a1)
    h2 = _gcn_layer(h1, adj2d, W2, b2, a2)
    return h2[None]


# speedup vs baseline: 1.0821x; 1.0821x over previous
import jax
import jax.numpy as jnp
from jax import lax
from jax.experimental import pallas as pl


def _feat_mm_body(x_ref, w_ref, y_ref):
    y_ref[...] = lax.dot_general(
        x_ref[...], w_ref[...], (((1,), (1,)), ((), ())),
        preferred_element_type=jnp.float32)


_N_STREAMS = 2


def _agg_body(a0, a1, y_ref, b_ref, al_ref, o_ref):
    alpha = al_ref[0, 0]
    for q, a_ref in enumerate((a0, a1)):
        acc = lax.dot_general(
            a_ref[0], y_ref[...], (((1,), (0,)), ((), ())),
            preferred_element_type=jnp.float32)
        h = acc + b_ref[...]
        o_ref[q] = jnp.where(h >= 0.0, h, alpha * h)


def _gcn_layer(x, adj2d, w, b, alpha, bm_feat=2000, bm_agg=200):
    n, d_in = x.shape
    d_out = w.shape[0]
    y = pl.pallas_call(
        _feat_mm_body,
        grid=(n // bm_feat,),
        in_specs=[
            pl.BlockSpec((bm_feat, d_in), lambda i: (i, 0)),
            pl.BlockSpec((d_out, d_in), lambda i: (0, 0)),
        ],
        out_specs=pl.BlockSpec((bm_feat, d_out), lambda i: (i, 0)),
        out_shape=jax.ShapeDtypeStruct((n, d_out), jnp.float32),
    )(x, w)
    ns = _N_STREAMS
    rows_per_stream = n // ns
    adj3 = adj2d.reshape(ns, rows_per_stream, n)
    adj_specs = [
        pl.BlockSpec((1, bm_agg, n), lambda i, q=q: (q, i, 0))
        for q in range(ns)
    ]
    h = pl.pallas_call(
        _agg_body,
        grid=(rows_per_stream // bm_agg,),
        in_specs=adj_specs + [
            pl.BlockSpec((n, d_out), lambda i: (0, 0)),
            pl.BlockSpec((1, d_out), lambda i: (0, 0)),
            pl.BlockSpec((1, 1), lambda i: (0, 0)),
        ],
        out_specs=pl.BlockSpec((ns, bm_agg, d_out), lambda i: (0, i, 0)),
        out_shape=jax.ShapeDtypeStruct((ns, rows_per_stream, d_out), jnp.float32),
    )(*([adj3] * ns), y, b.reshape(1, -1), alpha.reshape(1, 1))
    return h.reshape(n, d_out)


def kernel(features, seq1, adj, b1, W1, a1, b2, W2, a2, sparse):
    del seq1, sparse
    x = features[0]
    adj2d = adj[0]
    h1 = _gcn_layer(x, adj2d, W1, b1, a1)
    h2 = _gcn_layer(h1, adj2d, W2, b2, a2)
    return h2[None]
